# baseline (speedup 1.0000x reference)
import functools

import jax
import jax.numpy as jnp
from jax import lax
from jax.experimental import pallas as pl
from jax.experimental.pallas import tpu as pltpu
from jax.experimental.pallas import tpu_sc as plsc

_NE = 8192
_CD = 64
_CIN = 768
_M = 9216
_MB = 1024
_KB = 2048


def _bdot(a, b, dims=(((1,), (0,)), ((), ()))):
    return lax.dot_general(a.astype(jnp.bfloat16), b.astype(jnp.bfloat16),
                           dims, preferred_element_type=jnp.float32)


def _dist_argmin_body(z_ref, emb_ref, wct_ref, bct_ref, wc_ref, bc_ref,
                      idx_ref, zc_ref, cb_ref, cn_ref):
    i = pl.program_id(0)

    @pl.when(i == 0)
    def _init():
        cb = _bdot(emb_ref[...], wct_ref[...]) + bct_ref[...]
        cb_ref[...] = jnp.concatenate([cb, jnp.zeros_like(cb)], axis=1)
        cn_ref[...] = jnp.sum(cb * cb, axis=1)[None, :]

    zc = _bdot(z_ref[...], wc_ref[...]) + bc_ref[...]
    zcp = jnp.concatenate([zc, jnp.zeros_like(zc)], axis=1)
    zc_ref[...] = zcp
    zcn = -2.0 * zcp

    def chunk(k, carry):
        bmin, bidx = carry
        cbc = cb_ref[pl.ds(k * _KB, _KB), :]
        acc = _bdot(zcn, cbc, (((1,), (1,)), ((), ())))
        s = cn_ref[:, pl.ds(k * _KB, _KB)] + acc
        m = jnp.min(s, axis=1, keepdims=True)
        iota = lax.broadcasted_iota(jnp.int32, s.shape, 1)
        ii = jnp.min(jnp.where(s == m, iota, _NE), axis=1,
                     keepdims=True) + k * _KB
        take = m < bmin
        return jnp.where(take, m, bmin), jnp.where(take, ii, bidx)

    init = (jnp.full((_MB, 1), jnp.inf, jnp.float32),
            jnp.zeros((_MB, 1), jnp.int32))
    _, bidx = lax.fori_loop(0, _NE // _KB, chunk, init)
    idx_ref[...] = bidx[:, 0]


def _tc_dist_argmin(z2, emb, wct, bct2, wc, bc2):
    grid = _M // _MB
    return pl.pallas_call(
        _dist_argmin_body,
        grid=(grid,),
        in_specs=[
            pl.BlockSpec((_MB, _CIN), lambda i: (i, 0)),
            pl.BlockSpec((_NE, _CD), lambda i: (0, 0)),
            pl.BlockSpec((_CD, _CD), lambda i: (0, 0)),
            pl.BlockSpec((1, _CD), lambda i: (0, 0)),
            pl.BlockSpec((_CIN, _CD), lambda i: (0, 0)),
            pl.BlockSpec((1, _CD), lambda i: (0, 0)),
        ],
        out_specs=[
            pl.BlockSpec((_MB,), lambda i: (i,)),
            pl.BlockSpec((_MB, 2 * _CD), lambda i: (i, 0)),
            pl.BlockSpec((_NE, 2 * _CD), lambda i: (0, 0)),
        ],
        out_shape=[
            jax.ShapeDtypeStruct((_M,), jnp.int32),
            jax.ShapeDtypeStruct((_M, 2 * _CD), jnp.float32),
            jax.ShapeDtypeStruct((_NE, 2 * _CD), jnp.float32),
        ],
        scratch_shapes=[pltpu.VMEM((1, _NE), jnp.float32)],
    )(z2, emb, wct, bct2, wc, bc2)


def _expand_body(zq_ref, zc_ref, wx_ref, bx_ref, out_ref, loss_ref, acc_ref):
    i = pl.program_id(0)
    zq = zq_ref[...]
    out_ref[...] = _bdot(zq, wx_ref[...]) + bx_ref[...]
    d = zq - zc_ref[...]
    part = jnp.sum(d * d)

    @pl.when(i == 0)
    def _first():
        acc_ref[0, 0] = part

    @pl.when(i != 0)
    def _rest():
        acc_ref[0, 0] = acc_ref[0, 0] + part

    @pl.when(i == pl.num_programs(0) - 1)
    def _last():
        loss_ref[0, 0] = 3.0 * acc_ref[0, 0] / float(_M * _CD)


def _tc_expand(zq, zc, wx, bx2):
    grid = _M // _MB
    return pl.pallas_call(
        _expand_body,
        grid=(grid,),
        in_specs=[
            pl.BlockSpec((_MB, 2 * _CD), lambda i: (i, 0)),
            pl.BlockSpec((_MB, 2 * _CD), lambda i: (i, 0)),
            pl.BlockSpec((2 * _CD, _CIN), lambda i: (0, 0)),
            pl.BlockSpec((1, _CIN), lambda i: (0, 0)),
        ],
        out_specs=[
            pl.BlockSpec((_MB, _CIN), lambda i: (i, 0)),
            pl.BlockSpec((1, 1), lambda i: (0, 0), memory_space=pltpu.SMEM),
        ],
        out_shape=[
            jax.ShapeDtypeStruct((_M, _CIN), jnp.float32),
            jax.ShapeDtypeStruct((1, 1), jnp.float32),
        ],
        scratch_shapes=[pltpu.SMEM((1, 1), jnp.float32)],
    )(zq, zc, wx, bx2)


def _sc_gather(table, idx):
    info = plsc.get_sparse_core_info()
    nc, ns = info.num_cores, info.num_subcores
    nw = nc * ns
    bpw = _M // nw
    mesh = plsc.VectorSubcoreMesh(core_axis_name="c", subcore_axis_name="s")

    @functools.partial(
        pl.kernel, mesh=mesh,
        out_type=jax.ShapeDtypeStruct((_M, 2 * _CD), jnp.float32),
        scratch_types=[
            pltpu.VMEM((bpw,), jnp.int32),
            pltpu.VMEM((bpw, 2 * _CD), jnp.float32),
            pltpu.SemaphoreType.DMA,
        ],
    )
    def gather(table_hbm, idx_hbm, out_hbm, idx_v, rows_v, sem):
        wid = lax.axis_index("s") * nc + lax.axis_index("c")
        base = wid * bpw
        pltpu.sync_copy(idx_hbm.at[pl.ds(base, bpw)], idx_v)
        pltpu.async_copy(table_hbm.at[idx_v], rows_v, sem).wait()
        pltpu.sync_copy(rows_v, out_hbm.at[pl.ds(base, bpw)])

    return gather(table, idx)


def kernel(z, emb, W_ct, b_ct, W_c, b_c, W_x, b_x):
    B, T, CIN = z.shape
    z2 = z.reshape(-1, CIN)
    idx, zc, cb = _tc_dist_argmin(z2, emb, W_ct, b_ct.reshape(1, -1),
                                  W_c, b_c.reshape(1, -1))
    zq = _sc_gather(cb, idx)
    wxp = jnp.concatenate([W_x, jnp.zeros_like(W_x)], axis=0)
    out2, loss = _tc_expand(zq, zc, wxp, b_x.reshape(1, -1))
    return out2.reshape(B, T, CIN), loss[0, 0]


# speedup vs baseline: 1.2805x; 1.2805x over previous
import functools

import jax
import jax.numpy as jnp
from jax import lax
from jax.experimental import pallas as pl
from jax.experimental.pallas import tpu as pltpu
from jax.experimental.pallas import tpu_sc as plsc

_NE = 8192
_CD = 64
_CIN = 768
_M = 9216
_MB = 1024
_KB = 2048


def _bdot(a, b, dims=(((1,), (0,)), ((), ()))):
    return lax.dot_general(a.astype(jnp.bfloat16), b.astype(jnp.bfloat16),
                           dims, preferred_element_type=jnp.float32)


def _dist_argmin_body(z_ref, emb_ref, wct_ref, bct_ref, wc_ref, bc_ref,
                      idx_ref, zc_ref, cb_ref, cn_ref):
    i = pl.program_id(0)

    @pl.when(i == 0)
    def _init():
        cb = _bdot(emb_ref[...], wct_ref[...]) + bct_ref[...]
        cb_ref[...] = jnp.concatenate([cb, jnp.zeros_like(cb)], axis=1)
        cn_ref[...] = jnp.sum(cb * cb, axis=1)[None, :]

    zc = _bdot(z_ref[...], wc_ref[...]) + bc_ref[...]
    zcp = jnp.concatenate([zc, jnp.zeros_like(zc)], axis=1)
    zc_ref[...] = zcp
    zcn = -2.0 * zcp

    def chunk(k, carry):
        bmin, bidx = carry
        cbc = cb_ref[pl.ds(k * _KB, _KB), :]
        acc = _bdot(zcn, cbc, (((1,), (1,)), ((), ())))
        s = cn_ref[:, pl.ds(k * _KB, _KB)] + acc
        L = 128
        iota0 = lax.broadcasted_iota(jnp.int32, (_MB, L), 1)
        minv = s[:, :L]
        mini = iota0
        for j in range(1, _KB // L):
            sj = s[:, j * L:(j + 1) * L]
            c = sj < minv
            minv = jnp.where(c, sj, minv)
            mini = jnp.where(c, iota0 + j * L, mini)
        m = jnp.min(minv, axis=1, keepdims=True)
        ii = jnp.min(jnp.where(minv == m, mini, _NE), axis=1,
                     keepdims=True) + k * _KB
        take = m < bmin
        return jnp.where(take, m, bmin), jnp.where(take, ii, bidx)

    init = (jnp.full((_MB, 1), jnp.inf, jnp.float32),
            jnp.zeros((_MB, 1), jnp.int32))
    _, bidx = lax.fori_loop(0, _NE // _KB, chunk, init)
    idx_ref[...] = bidx[:, 0]


def _tc_dist_argmin(z2, emb, wct, bct2, wc, bc2):
    grid = _M // _MB
    return pl.pallas_call(
        _dist_argmin_body,
        grid=(grid,),
        in_specs=[
            pl.BlockSpec((_MB, _CIN), lambda i: (i, 0)),
            pl.BlockSpec((_NE, _CD), lambda i: (0, 0)),
            pl.BlockSpec((_CD, _CD), lambda i: (0, 0)),
            pl.BlockSpec((1, _CD), lambda i: (0, 0)),
            pl.BlockSpec((_CIN, _CD), lambda i: (0, 0)),
            pl.BlockSpec((1, _CD), lambda i: (0, 0)),
        ],
        out_specs=[
            pl.BlockSpec((_MB,), lambda i: (i,)),
            pl.BlockSpec((_MB, 2 * _CD), lambda i: (i, 0)),
            pl.BlockSpec((_NE, 2 * _CD), lambda i: (0, 0)),
        ],
        out_shape=[
            jax.ShapeDtypeStruct((_M,), jnp.int32),
            jax.ShapeDtypeStruct((_M, 2 * _CD), jnp.float32),
            jax.ShapeDtypeStruct((_NE, 2 * _CD), jnp.float32),
        ],
        scratch_shapes=[pltpu.VMEM((1, _NE), jnp.float32)],
    )(z2, emb, wct, bct2, wc, bc2)


def _expand_body(zq_ref, zc_ref, wx_ref, bx_ref, out_ref, loss_ref, acc_ref):
    i = pl.program_id(0)
    zq = zq_ref[...]
    out_ref[...] = _bdot(zq, wx_ref[...]) + bx_ref[...]
    d = zq - zc_ref[...]
    part = jnp.sum(d * d)

    @pl.when(i == 0)
    def _first():
        acc_ref[0, 0] = part

    @pl.when(i != 0)
    def _rest():
        acc_ref[0, 0] = acc_ref[0, 0] + part

    @pl.when(i == pl.num_programs(0) - 1)
    def _last():
        loss_ref[0, 0] = 3.0 * acc_ref[0, 0] / float(_M * _CD)


def _tc_expand(zq, zc, wx, bx2):
    grid = _M // _MB
    return pl.pallas_call(
        _expand_body,
        grid=(grid,),
        in_specs=[
            pl.BlockSpec((_MB, 2 * _CD), lambda i: (i, 0)),
            pl.BlockSpec((_MB, 2 * _CD), lambda i: (i, 0)),
            pl.BlockSpec((2 * _CD, _CIN), lambda i: (0, 0)),
            pl.BlockSpec((1, _CIN), lambda i: (0, 0)),
        ],
        out_specs=[
            pl.BlockSpec((_MB, _CIN), lambda i: (i, 0)),
            pl.BlockSpec((1, 1), lambda i: (0, 0), memory_space=pltpu.SMEM),
        ],
        out_shape=[
            jax.ShapeDtypeStruct((_M, _CIN), jnp.float32),
            jax.ShapeDtypeStruct((1, 1), jnp.float32),
        ],
        scratch_shapes=[pltpu.SMEM((1, 1), jnp.float32)],
    )(zq, zc, wx, bx2)


def _sc_gather(table, idx):
    info = plsc.get_sparse_core_info()
    nc, ns = info.num_cores, info.num_subcores
    nw = nc * ns
    bpw = _M // nw
    mesh = plsc.VectorSubcoreMesh(core_axis_name="c", subcore_axis_name="s")

    @functools.partial(
        pl.kernel, mesh=mesh,
        out_type=jax.ShapeDtypeStruct((_M, 2 * _CD), jnp.float32),
        scratch_types=[
            pltpu.VMEM((bpw,), jnp.int32),
            pltpu.VMEM((bpw, 2 * _CD), jnp.float32),
            pltpu.SemaphoreType.DMA,
        ],
    )
    def gather(table_hbm, idx_hbm, out_hbm, idx_v, rows_v, sem):
        wid = lax.axis_index("s") * nc + lax.axis_index("c")
        base = wid * bpw
        pltpu.sync_copy(idx_hbm.at[pl.ds(base, bpw)], idx_v)
        pltpu.async_copy(table_hbm.at[idx_v], rows_v, sem).wait()
        pltpu.sync_copy(rows_v, out_hbm.at[pl.ds(base, bpw)])

    return gather(table, idx)


def kernel(z, emb, W_ct, b_ct, W_c, b_c, W_x, b_x):
    B, T, CIN = z.shape
    z2 = z.reshape(-1, CIN)
    idx, zc, cb = _tc_dist_argmin(z2, emb, W_ct, b_ct.reshape(1, -1),
                                  W_c, b_c.reshape(1, -1))
    zq = _sc_gather(cb, idx)
    wxp = jnp.concatenate([W_x, jnp.zeros_like(W_x)], axis=0)
    out2, loss = _tc_expand(zq, zc, wxp, b_x.reshape(1, -1))
    return out2.reshape(B, T, CIN), loss[0, 0]
